# baseline (speedup 1.0000x reference)
import functools

import jax
import jax.numpy as jnp
import numpy as np
from jax.experimental import pallas as pl
from jax.experimental.pallas import tpu as pltpu

FRAG = 32
G_PER_HEAD = 3
PROTO_COUNT = 16
TOP_U = 8
TOPK_FRAC = 0.2
W_MEAN, W_MAX, W_TOPK, W_STD = 1.0, 0.6, 0.4, 0.2

T_Q = 512
GPAD = 8


def _normalize_safe(x, eps=1e-6):
    n = jnp.linalg.norm(x, axis=-1, keepdims=True)
    return x / jnp.maximum(n, eps)


def _fused_kernel(qp_ref, q_ref, k_ref, v_ref, o_ref, g_s, *, S, D, T, L, P, U):
    h = pl.program_id(0)
    t = pl.program_id(1)

    @pl.when(t == 0)
    def _select():
        kh = k_ref[0, 0]
        nrm = jnp.sqrt(jnp.sum(kh * kh, axis=-1, keepdims=True))
        kb = kh / jnp.maximum(nrm, 1e-6)
        qp = qp_ref[0]
        smt = jax.nn.relu(jax.lax.dot_general(
            qp, kb, (((1,), (1,)), ((), ())),
            preferred_element_type=jnp.float32))

        sub = jax.lax.broadcasted_iota(jnp.int32, (P, S), 0)
        mean = jnp.mean(smt, axis=0, keepdims=True)
        kq = max(1, int(round(P * TOPK_FRAC)))
        cur = smt
        s3 = jnp.zeros((1, S), jnp.float32)
        mx = None
        for _ in range(kq):
            mi = jnp.max(cur, axis=0, keepdims=True)
            if mx is None:
                mx = mi
            first = jnp.min(jnp.where(cur == mi, sub, P), axis=0,
                            keepdims=True)
            s3 = s3 + mi
            cur = jnp.where(sub == first, -jnp.inf, cur)
        topk_mean = s3 / float(kq)
        dev = smt - mean
        std = jnp.sqrt(jnp.sum(dev * dev, axis=0, keepdims=True) / (P - 1))
        u = (W_MEAN * mean + W_MAX * mx + W_TOPK * topk_mean
             + W_STD * std)

        col = jax.lax.broadcasted_iota(jnp.int32, (1, S), 1)
        val = u
        top_idx = []
        for _ in range(U):
            big = jnp.max(val, axis=1, keepdims=True)
            idxv = jnp.min(jnp.where(val == big, col, S), axis=1,
                           keepdims=True)
            top_idx.append(idxv)
            val = jnp.where(col == idxv, -1e9, val)
        tvec = jnp.concatenate(top_idx, axis=1)
        idx_s = [jnp.min(iv) for iv in top_idx]
        kraw = jnp.concatenate(
            [k_ref[0, 0, pl.ds(i_, 1), :] for i_ in idx_s], axis=0)
        n8 = jnp.sqrt(jnp.sum(kraw * kraw, axis=-1, keepdims=True))
        kb8 = kraw / jnp.maximum(n8, 1e-6)
        ssub = jax.nn.relu(jax.lax.dot_general(
            qp, kb8, (((1,), (1,)), ((), ())),
            preferred_element_type=jnp.float32))

        col_u = jax.lax.broadcasted_iota(jnp.int32, (1, U), 1)
        col_pu = jax.lax.broadcasted_iota(jnp.int32, (P, U), 1)
        m = jnp.zeros((P, 1), jnp.float32)
        blocked = jnp.zeros((1, U), jnp.bool_)
        for r in range(G_PER_HEAD):
            gains = jnp.sum(jax.nn.relu(ssub - m), axis=0, keepdims=True)
            gains = jnp.where(blocked, -1e9, gains)
            gbig = jnp.max(gains, axis=1, keepdims=True)
            j = jnp.min(jnp.where(gains == gbig, col_u, U), axis=1,
                        keepdims=True)
            g_s[0, r] = jnp.min(jnp.where(col_u == j, tvec, S))
            blocked = blocked | (col_u == j)
            picked = jnp.max(jnp.where(col_pu == j, ssub, -jnp.inf),
                             axis=1, keepdims=True)
            m = jnp.maximum(m, picked)

    t0 = t * T
    base = jnp.clip(t0 - FRAG // 2, 0, S - L)
    scale = 1.0 / np.sqrt(D)

    qb = q_ref[0, 0] * scale
    ks = k_ref[0, 0, pl.ds(base, L), :]
    vs = v_ref[0, 0, pl.ds(base, L), :]

    scores = jax.lax.dot_general(
        qb, ks, (((1,), (1,)), ((), ())),
        preferred_element_type=jnp.float32)

    t_abs = t0 + jax.lax.broadcasted_iota(jnp.int32, (T, L), 0)
    j_abs = base + jax.lax.broadcasted_iota(jnp.int32, (T, L), 1)
    start = jnp.clip(t_abs - FRAG // 2, 0, S - FRAG)
    in_band = (j_abs >= start) & (j_abs < start + FRAG)
    scores = jnp.where(in_band, scores, -1e30)

    rows_k = [k_ref[0, 0, pl.ds(g_s[0, g], 1), :] for g in range(G_PER_HEAD)]
    rows_v = [v_ref[0, 0, pl.ds(g_s[0, g], 1), :] for g in range(G_PER_HEAD)]
    pad = jnp.zeros((GPAD - G_PER_HEAD, D), jnp.float32)
    kg = jnp.concatenate(rows_k + [pad], axis=0)
    vg = jnp.concatenate(rows_v + [pad], axis=0)

    gscores = jax.lax.dot_general(
        qb, kg, (((1,), (1,)), ((), ())),
        preferred_element_type=jnp.float32)
    gcol = jax.lax.broadcasted_iota(jnp.int32, (T, GPAD), 1)
    gscores = jnp.where(gcol < G_PER_HEAD, gscores, -1e30)

    mrow = jnp.maximum(jnp.max(scores, axis=-1, keepdims=True),
                       jnp.max(gscores, axis=-1, keepdims=True))
    pw = jnp.exp(scores - mrow)
    pg = jnp.exp(gscores - mrow)
    denom = (jnp.sum(pw, axis=-1, keepdims=True) +
             jnp.sum(pg, axis=-1, keepdims=True))

    out = (jax.lax.dot_general(pw, vs, (((1,), (0,)), ((), ())),
                               preferred_element_type=jnp.float32) +
           jax.lax.dot_general(pg, vg, (((1,), (0,)), ((), ())),
                               preferred_element_type=jnp.float32))
    o_ref[0, 0] = out / denom


def kernel(q, k, v):
    B, H, S, D = q.shape
    P = min(PROTO_COUNT, S)
    U = max(G_PER_HEAD, min(TOP_U, S))
    idxp = np.round(np.linspace(0.0, S - 1, P)).astype(np.int32)
    Qp = _normalize_safe(q.mean(axis=0)[:, idxp, :])

    T = T_Q
    L = T + FRAG

    out = pl.pallas_call(
        functools.partial(_fused_kernel, S=S, D=D, T=T, L=L, P=P, U=U),
        grid=(H, S // T),
        in_specs=[
            pl.BlockSpec((1, P, D), lambda h, t: (h, 0, 0)),
            pl.BlockSpec((1, 1, T, D), lambda h, t: (0, h, t, 0)),
            pl.BlockSpec((1, 1, S, D), lambda h, t: (0, h, 0, 0)),
            pl.BlockSpec((1, 1, S, D), lambda h, t: (0, h, 0, 0)),
        ],
        out_specs=pl.BlockSpec((1, 1, T, D), lambda h, t: (0, h, t, 0)),
        out_shape=jax.ShapeDtypeStruct((B, H, S, D), jnp.float32),
        scratch_shapes=[pltpu.SMEM((1, GPAD), jnp.int32)],
        compiler_params=pltpu.CompilerParams(
            dimension_semantics=("parallel", "arbitrary")),
    )(Qp, q, k, v)

    return out


# speedup vs baseline: 1.1177x; 1.1177x over previous
import functools

import jax
import jax.numpy as jnp
import numpy as np
from jax.experimental import pallas as pl
from jax.experimental.pallas import tpu as pltpu

FRAG = 32
G_PER_HEAD = 3
PROTO_COUNT = 16
TOP_U = 8
TOPK_FRAC = 0.2
W_MEAN, W_MAX, W_TOPK, W_STD = 1.0, 0.6, 0.4, 0.2

T_Q = 512
HPB = 2
GPAD = 8


def _normalize_safe(x, eps=1e-6):
    n = jnp.linalg.norm(x, axis=-1, keepdims=True)
    return x / jnp.maximum(n, eps)


def _route_one_head(kh, qp, get_krow, *, S, P, U):
    nrm = jnp.sqrt(jnp.sum(kh * kh, axis=-1, keepdims=True))
    kb = kh / jnp.maximum(nrm, 1e-6)
    smt = jax.nn.relu(jax.lax.dot_general(
        qp, kb, (((1,), (1,)), ((), ())),
        preferred_element_type=jnp.float32))

    sub = jax.lax.broadcasted_iota(jnp.int32, (P, S), 0)
    mean = jnp.mean(smt, axis=0, keepdims=True)
    kq = max(1, int(round(P * TOPK_FRAC)))
    cur = smt
    s3 = jnp.zeros((1, S), jnp.float32)
    mx = None
    for _ in range(kq):
        mi = jnp.max(cur, axis=0, keepdims=True)
        if mx is None:
            mx = mi
        first = jnp.min(jnp.where(cur == mi, sub, P), axis=0, keepdims=True)
        s3 = s3 + mi
        cur = jnp.where(sub == first, -jnp.inf, cur)
    topk_mean = s3 / float(kq)
    dev = smt - mean
    std = jnp.sqrt(jnp.sum(dev * dev, axis=0, keepdims=True) / (P - 1))
    u = W_MEAN * mean + W_MAX * mx + W_TOPK * topk_mean + W_STD * std

    col = jax.lax.broadcasted_iota(jnp.int32, (1, S), 1)
    val = u
    top_idx = []
    for _ in range(U):
        big = jnp.max(val, axis=1, keepdims=True)
        idxv = jnp.min(jnp.where(val == big, col, S), axis=1,
                       keepdims=True)
        top_idx.append(idxv)
        val = jnp.where(col == idxv, -1e9, val)
    tvec = jnp.concatenate(top_idx, axis=1)
    idx_s = [jnp.min(iv) for iv in top_idx]
    kraw = jnp.concatenate([get_krow(i_) for i_ in idx_s], axis=0)
    n8 = jnp.sqrt(jnp.sum(kraw * kraw, axis=-1, keepdims=True))
    kb8 = kraw / jnp.maximum(n8, 1e-6)
    ssub = jax.nn.relu(jax.lax.dot_general(
        qp, kb8, (((1,), (1,)), ((), ())),
        preferred_element_type=jnp.float32))

    col_u = jax.lax.broadcasted_iota(jnp.int32, (1, U), 1)
    col_pu = jax.lax.broadcasted_iota(jnp.int32, (P, U), 1)
    m = jnp.zeros((P, 1), jnp.float32)
    blocked = jnp.zeros((1, U), jnp.bool_)
    chosen = []
    for r in range(G_PER_HEAD):
        gains = jnp.sum(jax.nn.relu(ssub - m), axis=0, keepdims=True)
        gains = jnp.where(blocked, -1e9, gains)
        gbig = jnp.max(gains, axis=1, keepdims=True)
        j = jnp.min(jnp.where(gains == gbig, col_u, U), axis=1,
                    keepdims=True)
        chosen.append(jnp.min(jnp.where(col_u == j, tvec, S)))
        blocked = blocked | (col_u == j)
        picked = jnp.max(jnp.where(col_pu == j, ssub, -jnp.inf),
                         axis=1, keepdims=True)
        m = jnp.maximum(m, picked)
    return chosen


def _attend_one_head(qb, k_slice, v_slice, get_row, g_idx, t0, base,
                     *, S, T, L, D):
    ks = k_slice
    vs = v_slice
    scores = jax.lax.dot_general(
        qb, ks, (((1,), (1,)), ((), ())),
        preferred_element_type=jnp.float32)

    t_abs = t0 + jax.lax.broadcasted_iota(jnp.int32, (T, L), 0)
    j_abs = base + jax.lax.broadcasted_iota(jnp.int32, (T, L), 1)
    start = jnp.clip(t_abs - FRAG // 2, 0, S - FRAG)
    in_band = (j_abs >= start) & (j_abs < start + FRAG)
    scores = jnp.where(in_band, scores, -1e30)

    rows_k, rows_v = zip(*[get_row(g_idx[g]) for g in range(G_PER_HEAD)])
    pad = jnp.zeros((GPAD - G_PER_HEAD, D), jnp.float32)
    kg = jnp.concatenate(list(rows_k) + [pad], axis=0)
    vg = jnp.concatenate(list(rows_v) + [pad], axis=0)

    gscores = jax.lax.dot_general(
        qb, kg, (((1,), (1,)), ((), ())),
        preferred_element_type=jnp.float32)
    gcol = jax.lax.broadcasted_iota(jnp.int32, (T, GPAD), 1)
    gscores = jnp.where(gcol < G_PER_HEAD, gscores, -1e30)

    mrow = jnp.maximum(jnp.max(scores, axis=-1, keepdims=True),
                       jnp.max(gscores, axis=-1, keepdims=True))
    pw = jnp.exp(scores - mrow)
    pg = jnp.exp(gscores - mrow)
    denom = (jnp.sum(pw, axis=-1, keepdims=True) +
             jnp.sum(pg, axis=-1, keepdims=True))

    return (jax.lax.dot_general(pw, vs, (((1,), (0,)), ((), ())),
                                preferred_element_type=jnp.float32) +
            jax.lax.dot_general(pg, vg, (((1,), (0,)), ((), ())),
                                preferred_element_type=jnp.float32)) / denom


def _fused_kernel(qp_ref, q_ref, k_ref, v_ref, o_ref, g_s,
                  *, S, D, T, L, P, U):
    t = pl.program_id(1)

    @pl.when(t == 0)
    def _select():
        for hh in range(HPB):
            chosen = _route_one_head(
                k_ref[0, hh], qp_ref[hh],
                lambda i_, hh=hh: k_ref[0, hh, pl.ds(i_, 1), :],
                S=S, P=P, U=U)
            for r in range(G_PER_HEAD):
                g_s[hh, r] = chosen[r]

    t0 = t * T
    base = jnp.clip(t0 - FRAG // 2, 0, S - L)
    scale = 1.0 / np.sqrt(D)
    for hh in range(HPB):
        o_ref[0, hh] = _attend_one_head(
            q_ref[0, hh] * scale,
            k_ref[0, hh, pl.ds(base, L), :],
            v_ref[0, hh, pl.ds(base, L), :],
            lambda g, hh=hh: (k_ref[0, hh, pl.ds(g, 1), :],
                              v_ref[0, hh, pl.ds(g, 1), :]),
            [g_s[hh, g] for g in range(G_PER_HEAD)],
            t0, base, S=S, T=T, L=L, D=D)


def kernel(q, k, v):
    B, H, S, D = q.shape
    P = min(PROTO_COUNT, S)
    U = max(G_PER_HEAD, min(TOP_U, S))
    idxp = np.round(np.linspace(0.0, S - 1, P)).astype(np.int32)
    Qp = _normalize_safe(q.mean(axis=0)[:, idxp, :])

    T = T_Q
    L = T + FRAG

    out = pl.pallas_call(
        functools.partial(_fused_kernel, S=S, D=D, T=T, L=L, P=P, U=U),
        grid=(H // HPB, S // T),
        in_specs=[
            pl.BlockSpec((HPB, P, D), lambda h, t: (h, 0, 0)),
            pl.BlockSpec((1, HPB, T, D), lambda h, t: (0, h, t, 0)),
            pl.BlockSpec((1, HPB, S, D), lambda h, t: (0, h, 0, 0)),
            pl.BlockSpec((1, HPB, S, D), lambda h, t: (0, h, 0, 0)),
        ],
        out_specs=pl.BlockSpec((1, HPB, T, D), lambda h, t: (0, h, t, 0)),
        out_shape=jax.ShapeDtypeStruct((B, H, S, D), jnp.float32),
        scratch_shapes=[pltpu.SMEM((HPB, GPAD), jnp.int32)],
        compiler_params=pltpu.CompilerParams(
            dimension_semantics=("parallel", "arbitrary")),
    )(Qp, q, k, v)

    return out


# speedup vs baseline: 1.1971x; 1.0710x over previous
import functools

import jax
import jax.numpy as jnp
import numpy as np
from jax.experimental import pallas as pl
from jax.experimental.pallas import tpu as pltpu

FRAG = 32
G_PER_HEAD = 3
PROTO_COUNT = 16
TOP_U = 8
TOPK_FRAC = 0.2
W_MEAN, W_MAX, W_TOPK, W_STD = 1.0, 0.6, 0.4, 0.2

T_Q = 512
HPB = 4
GPAD = 8


def _normalize_safe(x, eps=1e-6):
    n = jnp.linalg.norm(x, axis=-1, keepdims=True)
    return x / jnp.maximum(n, eps)


def _route_one_head(kh, qp, get_krow, *, S, P, U):
    nrm = jnp.sqrt(jnp.sum(kh * kh, axis=-1, keepdims=True))
    kb = kh / jnp.maximum(nrm, 1e-6)
    smt = jax.nn.relu(jax.lax.dot_general(
        qp, kb, (((1,), (1,)), ((), ())),
        preferred_element_type=jnp.float32))

    sub = jax.lax.broadcasted_iota(jnp.int32, (P, S), 0)
    mean = jnp.mean(smt, axis=0, keepdims=True)
    kq = max(1, int(round(P * TOPK_FRAC)))
    cur = smt
    s3 = jnp.zeros((1, S), jnp.float32)
    mx = None
    for _ in range(kq):
        mi = jnp.max(cur, axis=0, keepdims=True)
        if mx is None:
            mx = mi
        first = jnp.min(jnp.where(cur == mi, sub, P), axis=0, keepdims=True)
        s3 = s3 + mi
        cur = jnp.where(sub == first, -jnp.inf, cur)
    topk_mean = s3 / float(kq)
    dev = smt - mean
    std = jnp.sqrt(jnp.sum(dev * dev, axis=0, keepdims=True) / (P - 1))
    u = W_MEAN * mean + W_MAX * mx + W_TOPK * topk_mean + W_STD * std

    col = jax.lax.broadcasted_iota(jnp.int32, (1, S), 1)
    val = u
    top_idx = []
    for _ in range(U):
        big = jnp.max(val, axis=1, keepdims=True)
        idxv = jnp.min(jnp.where(val == big, col, S), axis=1,
                       keepdims=True)
        top_idx.append(idxv)
        val = jnp.where(col == idxv, -1e9, val)
    tvec = jnp.concatenate(top_idx, axis=1)
    idx_s = [jnp.min(iv) for iv in top_idx]
    kraw = jnp.concatenate([get_krow(i_) for i_ in idx_s], axis=0)
    n8 = jnp.sqrt(jnp.sum(kraw * kraw, axis=-1, keepdims=True))
    kb8 = kraw / jnp.maximum(n8, 1e-6)
    ssub = jax.nn.relu(jax.lax.dot_general(
        qp, kb8, (((1,), (1,)), ((), ())),
        preferred_element_type=jnp.float32))

    col_u = jax.lax.broadcasted_iota(jnp.int32, (1, U), 1)
    col_pu = jax.lax.broadcasted_iota(jnp.int32, (P, U), 1)
    m = jnp.zeros((P, 1), jnp.float32)
    blocked = jnp.zeros((1, U), jnp.bool_)
    chosen = []
    for r in range(G_PER_HEAD):
        gains = jnp.sum(jax.nn.relu(ssub - m), axis=0, keepdims=True)
        gains = jnp.where(blocked, -1e9, gains)
        gbig = jnp.max(gains, axis=1, keepdims=True)
        j = jnp.min(jnp.where(gains == gbig, col_u, U), axis=1,
                    keepdims=True)
        chosen.append(jnp.min(jnp.where(col_u == j, tvec, S)))
        blocked = blocked | (col_u == j)
        picked = jnp.max(jnp.where(col_pu == j, ssub, -jnp.inf),
                         axis=1, keepdims=True)
        m = jnp.maximum(m, picked)
    return chosen


def _attend_one_head(qb, k_slice, v_slice, get_row, g_idx, t0, base,
                     *, S, T, L, D):
    ks = k_slice
    vs = v_slice
    scores = jax.lax.dot_general(
        qb, ks, (((1,), (1,)), ((), ())),
        preferred_element_type=jnp.float32)

    t_abs = t0 + jax.lax.broadcasted_iota(jnp.int32, (T, L), 0)
    j_abs = base + jax.lax.broadcasted_iota(jnp.int32, (T, L), 1)
    start = jnp.clip(t_abs - FRAG // 2, 0, S - FRAG)
    in_band = (j_abs >= start) & (j_abs < start + FRAG)
    scores = jnp.where(in_band, scores, -1e30)

    rows_k, rows_v = zip(*[get_row(g_idx[g]) for g in range(G_PER_HEAD)])
    pad = jnp.zeros((GPAD - G_PER_HEAD, D), jnp.float32)
    kg = jnp.concatenate(list(rows_k) + [pad], axis=0)
    vg = jnp.concatenate(list(rows_v) + [pad], axis=0)

    gscores = jax.lax.dot_general(
        qb, kg, (((1,), (1,)), ((), ())),
        preferred_element_type=jnp.float32)
    gcol = jax.lax.broadcasted_iota(jnp.int32, (T, GPAD), 1)
    gscores = jnp.where(gcol < G_PER_HEAD, gscores, -1e30)

    mrow = jnp.maximum(jnp.max(scores, axis=-1, keepdims=True),
                       jnp.max(gscores, axis=-1, keepdims=True))
    pw = jnp.exp(scores - mrow)
    pg = jnp.exp(gscores - mrow)
    denom = (jnp.sum(pw, axis=-1, keepdims=True) +
             jnp.sum(pg, axis=-1, keepdims=True))

    return (jax.lax.dot_general(pw, vs, (((1,), (0,)), ((), ())),
                                preferred_element_type=jnp.float32) +
            jax.lax.dot_general(pg, vg, (((1,), (0,)), ((), ())),
                                preferred_element_type=jnp.float32)) / denom


def _fused_kernel(qp_ref, q_ref, k_ref, v_ref, o_ref, g_s,
                  *, S, D, T, L, P, U):
    t = pl.program_id(1)

    @pl.when(t == 0)
    def _select():
        for hh in range(HPB):
            chosen = _route_one_head(
                k_ref[0, hh], qp_ref[hh],
                lambda i_, hh=hh: k_ref[0, hh, pl.ds(i_, 1), :],
                S=S, P=P, U=U)
            for r in range(G_PER_HEAD):
                g_s[hh, r] = chosen[r]

    t0 = t * T
    base = jnp.clip(t0 - FRAG // 2, 0, S - L)
    scale = 1.0 / np.sqrt(D)
    for hh in range(HPB):
        o_ref[0, hh] = _attend_one_head(
            q_ref[0, hh] * scale,
            k_ref[0, hh, pl.ds(base, L), :],
            v_ref[0, hh, pl.ds(base, L), :],
            lambda g, hh=hh: (k_ref[0, hh, pl.ds(g, 1), :],
                              v_ref[0, hh, pl.ds(g, 1), :]),
            [g_s[hh, g] for g in range(G_PER_HEAD)],
            t0, base, S=S, T=T, L=L, D=D)


def kernel(q, k, v):
    B, H, S, D = q.shape
    P = min(PROTO_COUNT, S)
    U = max(G_PER_HEAD, min(TOP_U, S))
    idxp = np.round(np.linspace(0.0, S - 1, P)).astype(np.int32)
    Qp = _normalize_safe(q.mean(axis=0)[:, idxp, :])

    T = T_Q
    L = T + FRAG

    out = pl.pallas_call(
        functools.partial(_fused_kernel, S=S, D=D, T=T, L=L, P=P, U=U),
        grid=(H // HPB, S // T),
        in_specs=[
            pl.BlockSpec((HPB, P, D), lambda h, t: (h, 0, 0)),
            pl.BlockSpec((1, HPB, T, D), lambda h, t: (0, h, t, 0)),
            pl.BlockSpec((1, HPB, S, D), lambda h, t: (0, h, 0, 0)),
            pl.BlockSpec((1, HPB, S, D), lambda h, t: (0, h, 0, 0)),
        ],
        out_specs=pl.BlockSpec((1, HPB, T, D), lambda h, t: (0, h, t, 0)),
        out_shape=jax.ShapeDtypeStruct((B, H, S, D), jnp.float32),
        scratch_shapes=[pltpu.SMEM((HPB, GPAD), jnp.int32)],
        compiler_params=pltpu.CompilerParams(
            dimension_semantics=("parallel", "arbitrary")),
    )(Qp, q, k, v)

    return out


# speedup vs baseline: 1.2325x; 1.0296x over previous
import functools

import jax
import jax.numpy as jnp
import numpy as np
from jax.experimental import pallas as pl
from jax.experimental.pallas import tpu as pltpu

FRAG = 32
G_PER_HEAD = 3
PROTO_COUNT = 16
TOP_U = 8
TOPK_FRAC = 0.2
W_MEAN, W_MAX, W_TOPK, W_STD = 1.0, 0.6, 0.4, 0.2

T_Q = 512
HPB = 8
GPAD = 8


def _normalize_safe(x, eps=1e-6):
    n = jnp.linalg.norm(x, axis=-1, keepdims=True)
    return x / jnp.maximum(n, eps)


def _route_one_head(kh, qp, get_krow, *, S, P, U):
    nrm = jnp.sqrt(jnp.sum(kh * kh, axis=-1, keepdims=True))
    kb = kh / jnp.maximum(nrm, 1e-6)
    smt = jax.nn.relu(jax.lax.dot_general(
        qp, kb, (((1,), (1,)), ((), ())),
        preferred_element_type=jnp.float32))

    sub = jax.lax.broadcasted_iota(jnp.int32, (P, S), 0)
    mean = jnp.mean(smt, axis=0, keepdims=True)
    kq = max(1, int(round(P * TOPK_FRAC)))
    cur = smt
    s3 = jnp.zeros((1, S), jnp.float32)
    mx = None
    for _ in range(kq):
        mi = jnp.max(cur, axis=0, keepdims=True)
        if mx is None:
            mx = mi
        first = jnp.min(jnp.where(cur == mi, sub, P), axis=0, keepdims=True)
        s3 = s3 + mi
        cur = jnp.where(sub == first, -jnp.inf, cur)
    topk_mean = s3 / float(kq)
    dev = smt - mean
    std = jnp.sqrt(jnp.sum(dev * dev, axis=0, keepdims=True) / (P - 1))
    u = W_MEAN * mean + W_MAX * mx + W_TOPK * topk_mean + W_STD * std

    col = jax.lax.broadcasted_iota(jnp.int32, (1, S), 1)
    val = u
    top_idx = []
    for _ in range(U):
        big = jnp.max(val, axis=1, keepdims=True)
        idxv = jnp.min(jnp.where(val == big, col, S), axis=1,
                       keepdims=True)
        top_idx.append(idxv)
        val = jnp.where(col == idxv, -1e9, val)
    tvec = jnp.concatenate(top_idx, axis=1)
    idx_s = [jnp.min(iv) for iv in top_idx]
    kraw = jnp.concatenate([get_krow(i_) for i_ in idx_s], axis=0)
    n8 = jnp.sqrt(jnp.sum(kraw * kraw, axis=-1, keepdims=True))
    kb8 = kraw / jnp.maximum(n8, 1e-6)
    ssub = jax.nn.relu(jax.lax.dot_general(
        qp, kb8, (((1,), (1,)), ((), ())),
        preferred_element_type=jnp.float32))

    col_u = jax.lax.broadcasted_iota(jnp.int32, (1, U), 1)
    col_pu = jax.lax.broadcasted_iota(jnp.int32, (P, U), 1)
    m = jnp.zeros((P, 1), jnp.float32)
    blocked = jnp.zeros((1, U), jnp.bool_)
    chosen = []
    for r in range(G_PER_HEAD):
        gains = jnp.sum(jax.nn.relu(ssub - m), axis=0, keepdims=True)
        gains = jnp.where(blocked, -1e9, gains)
        gbig = jnp.max(gains, axis=1, keepdims=True)
        j = jnp.min(jnp.where(gains == gbig, col_u, U), axis=1,
                    keepdims=True)
        chosen.append(jnp.min(jnp.where(col_u == j, tvec, S)))
        blocked = blocked | (col_u == j)
        picked = jnp.max(jnp.where(col_pu == j, ssub, -jnp.inf),
                         axis=1, keepdims=True)
        m = jnp.maximum(m, picked)
    return chosen


def _attend_one_head(qb, k_slice, v_slice, get_row, g_idx, t0, base,
                     *, S, T, L, D):
    ks = k_slice
    vs = v_slice
    scores = jax.lax.dot_general(
        qb, ks, (((1,), (1,)), ((), ())),
        preferred_element_type=jnp.float32)

    t_abs = t0 + jax.lax.broadcasted_iota(jnp.int32, (T, L), 0)
    j_abs = base + jax.lax.broadcasted_iota(jnp.int32, (T, L), 1)
    start = jnp.clip(t_abs - FRAG // 2, 0, S - FRAG)
    in_band = (j_abs >= start) & (j_abs < start + FRAG)
    scores = jnp.where(in_band, scores, -1e30)

    rows_k, rows_v = zip(*[get_row(g_idx[g]) for g in range(G_PER_HEAD)])
    pad = jnp.zeros((GPAD - G_PER_HEAD, D), jnp.float32)
    kg = jnp.concatenate(list(rows_k) + [pad], axis=0)
    vg = jnp.concatenate(list(rows_v) + [pad], axis=0)

    gscores = jax.lax.dot_general(
        qb, kg, (((1,), (1,)), ((), ())),
        preferred_element_type=jnp.float32)
    gcol = jax.lax.broadcasted_iota(jnp.int32, (T, GPAD), 1)
    gscores = jnp.where(gcol < G_PER_HEAD, gscores, -1e30)

    mrow = jnp.maximum(jnp.max(scores, axis=-1, keepdims=True),
                       jnp.max(gscores, axis=-1, keepdims=True))
    pw = jnp.exp(scores - mrow)
    pg = jnp.exp(gscores - mrow)
    denom = (jnp.sum(pw, axis=-1, keepdims=True) +
             jnp.sum(pg, axis=-1, keepdims=True))

    return (jax.lax.dot_general(pw, vs, (((1,), (0,)), ((), ())),
                                preferred_element_type=jnp.float32) +
            jax.lax.dot_general(pg, vg, (((1,), (0,)), ((), ())),
                                preferred_element_type=jnp.float32)) / denom


def _fused_kernel(qp_ref, q_ref, k_ref, v_ref, o_ref, g_s,
                  *, S, D, T, L, P, U):
    t = pl.program_id(1)

    @pl.when(t == 0)
    def _select():
        for hh in range(HPB):
            chosen = _route_one_head(
                k_ref[0, hh], qp_ref[hh],
                lambda i_, hh=hh: k_ref[0, hh, pl.ds(i_, 1), :],
                S=S, P=P, U=U)
            for r in range(G_PER_HEAD):
                g_s[hh, r] = chosen[r]

    t0 = t * T
    base = jnp.clip(t0 - FRAG // 2, 0, S - L)
    scale = 1.0 / np.sqrt(D)
    for hh in range(HPB):
        o_ref[0, hh] = _attend_one_head(
            q_ref[0, hh] * scale,
            k_ref[0, hh, pl.ds(base, L), :],
            v_ref[0, hh, pl.ds(base, L), :],
            lambda g, hh=hh: (k_ref[0, hh, pl.ds(g, 1), :],
                              v_ref[0, hh, pl.ds(g, 1), :]),
            [g_s[hh, g] for g in range(G_PER_HEAD)],
            t0, base, S=S, T=T, L=L, D=D)


def kernel(q, k, v):
    B, H, S, D = q.shape
    P = min(PROTO_COUNT, S)
    U = max(G_PER_HEAD, min(TOP_U, S))
    idxp = np.round(np.linspace(0.0, S - 1, P)).astype(np.int32)
    Qp = _normalize_safe(q.mean(axis=0)[:, idxp, :])

    T = T_Q
    L = T + FRAG

    out = pl.pallas_call(
        functools.partial(_fused_kernel, S=S, D=D, T=T, L=L, P=P, U=U),
        grid=(H // HPB, S // T),
        in_specs=[
            pl.BlockSpec((HPB, P, D), lambda h, t: (h, 0, 0)),
            pl.BlockSpec((1, HPB, T, D), lambda h, t: (0, h, t, 0)),
            pl.BlockSpec((1, HPB, S, D), lambda h, t: (0, h, 0, 0)),
            pl.BlockSpec((1, HPB, S, D), lambda h, t: (0, h, 0, 0)),
        ],
        out_specs=pl.BlockSpec((1, HPB, T, D), lambda h, t: (0, h, t, 0)),
        out_shape=jax.ShapeDtypeStruct((B, H, S, D), jnp.float32),
        scratch_shapes=[pltpu.SMEM((HPB, GPAD), jnp.int32)],
        compiler_params=pltpu.CompilerParams(
            dimension_semantics=("parallel", "arbitrary")),
    )(Qp, q, k, v)

    return out


# speedup vs baseline: 1.2669x; 1.0279x over previous
import functools

import jax
import jax.numpy as jnp
import numpy as np
from jax.experimental import pallas as pl
from jax.experimental.pallas import tpu as pltpu

FRAG = 32
G_PER_HEAD = 3
PROTO_COUNT = 16
TOP_U = 8
TOPK_FRAC = 0.2
W_MEAN, W_MAX, W_TOPK, W_STD = 1.0, 0.6, 0.4, 0.2

T_Q = 256
HPB = 8
GPAD = 8


def _normalize_safe(x, eps=1e-6):
    n = jnp.linalg.norm(x, axis=-1, keepdims=True)
    return x / jnp.maximum(n, eps)


def _route_one_head(kh, qp, get_krow, *, S, P, U):
    nrm = jnp.sqrt(jnp.sum(kh * kh, axis=-1, keepdims=True))
    kb = kh / jnp.maximum(nrm, 1e-6)
    smt = jax.nn.relu(jax.lax.dot_general(
        qp, kb, (((1,), (1,)), ((), ())),
        preferred_element_type=jnp.float32))

    sub = jax.lax.broadcasted_iota(jnp.int32, (P, S), 0)
    mean = jnp.mean(smt, axis=0, keepdims=True)
    kq = max(1, int(round(P * TOPK_FRAC)))
    cur = smt
    s3 = jnp.zeros((1, S), jnp.float32)
    mx = None
    for _ in range(kq):
        mi = jnp.max(cur, axis=0, keepdims=True)
        if mx is None:
            mx = mi
        first = jnp.min(jnp.where(cur == mi, sub, P), axis=0, keepdims=True)
        s3 = s3 + mi
        cur = jnp.where(sub == first, -jnp.inf, cur)
    topk_mean = s3 / float(kq)
    dev = smt - mean
    std = jnp.sqrt(jnp.sum(dev * dev, axis=0, keepdims=True) / (P - 1))
    u = W_MEAN * mean + W_MAX * mx + W_TOPK * topk_mean + W_STD * std

    col = jax.lax.broadcasted_iota(jnp.int32, (1, S), 1)
    val = u
    top_idx = []
    for _ in range(U):
        big = jnp.max(val, axis=1, keepdims=True)
        idxv = jnp.min(jnp.where(val == big, col, S), axis=1,
                       keepdims=True)
        top_idx.append(idxv)
        val = jnp.where(col == idxv, -1e9, val)
    tvec = jnp.concatenate(top_idx, axis=1)
    idx_s = [jnp.min(iv) for iv in top_idx]
    kraw = jnp.concatenate([get_krow(i_) for i_ in idx_s], axis=0)
    n8 = jnp.sqrt(jnp.sum(kraw * kraw, axis=-1, keepdims=True))
    kb8 = kraw / jnp.maximum(n8, 1e-6)
    ssub = jax.nn.relu(jax.lax.dot_general(
        qp, kb8, (((1,), (1,)), ((), ())),
        preferred_element_type=jnp.float32))

    col_u = jax.lax.broadcasted_iota(jnp.int32, (1, U), 1)
    col_pu = jax.lax.broadcasted_iota(jnp.int32, (P, U), 1)
    m = jnp.zeros((P, 1), jnp.float32)
    blocked = jnp.zeros((1, U), jnp.bool_)
    chosen = []
    for r in range(G_PER_HEAD):
        gains = jnp.sum(jax.nn.relu(ssub - m), axis=0, keepdims=True)
        gains = jnp.where(blocked, -1e9, gains)
        gbig = jnp.max(gains, axis=1, keepdims=True)
        j = jnp.min(jnp.where(gains == gbig, col_u, U), axis=1,
                    keepdims=True)
        chosen.append(jnp.min(jnp.where(col_u == j, tvec, S)))
        blocked = blocked | (col_u == j)
        picked = jnp.max(jnp.where(col_pu == j, ssub, -jnp.inf),
                         axis=1, keepdims=True)
        m = jnp.maximum(m, picked)
    return chosen


def _attend_one_head(qb, k_slice, v_slice, get_row, g_idx, t0, base,
                     *, S, T, L, D):
    ks = k_slice
    vs = v_slice
    scores = jax.lax.dot_general(
        qb, ks, (((1,), (1,)), ((), ())),
        preferred_element_type=jnp.float32)

    t_abs = t0 + jax.lax.broadcasted_iota(jnp.int32, (T, L), 0)
    j_abs = base + jax.lax.broadcasted_iota(jnp.int32, (T, L), 1)
    start = jnp.clip(t_abs - FRAG // 2, 0, S - FRAG)
    in_band = (j_abs >= start) & (j_abs < start + FRAG)
    scores = jnp.where(in_band, scores, -1e30)

    rows_k, rows_v = zip(*[get_row(g_idx[g]) for g in range(G_PER_HEAD)])
    pad = jnp.zeros((GPAD - G_PER_HEAD, D), jnp.float32)
    kg = jnp.concatenate(list(rows_k) + [pad], axis=0)
    vg = jnp.concatenate(list(rows_v) + [pad], axis=0)

    gscores = jax.lax.dot_general(
        qb, kg, (((1,), (1,)), ((), ())),
        preferred_element_type=jnp.float32)
    gcol = jax.lax.broadcasted_iota(jnp.int32, (T, GPAD), 1)
    gscores = jnp.where(gcol < G_PER_HEAD, gscores, -1e30)

    mrow = jnp.maximum(jnp.max(scores, axis=-1, keepdims=True),
                       jnp.max(gscores, axis=-1, keepdims=True))
    pw = jnp.exp(scores - mrow)
    pg = jnp.exp(gscores - mrow)
    denom = (jnp.sum(pw, axis=-1, keepdims=True) +
             jnp.sum(pg, axis=-1, keepdims=True))

    return (jax.lax.dot_general(pw, vs, (((1,), (0,)), ((), ())),
                                preferred_element_type=jnp.float32) +
            jax.lax.dot_general(pg, vg, (((1,), (0,)), ((), ())),
                                preferred_element_type=jnp.float32)) / denom


def _fused_kernel(qp_ref, q_ref, k_ref, v_ref, o_ref, g_s,
                  *, S, D, T, L, P, U):
    t = pl.program_id(1)

    @pl.when(t == 0)
    def _select():
        for hh in range(HPB):
            chosen = _route_one_head(
                k_ref[0, hh], qp_ref[hh],
                lambda i_, hh=hh: k_ref[0, hh, pl.ds(i_, 1), :],
                S=S, P=P, U=U)
            for r in range(G_PER_HEAD):
                g_s[hh, r] = chosen[r]

    t0 = t * T
    base = jnp.clip(t0 - FRAG // 2, 0, S - L)
    scale = 1.0 / np.sqrt(D)
    for hh in range(HPB):
        o_ref[0, hh] = _attend_one_head(
            q_ref[0, hh] * scale,
            k_ref[0, hh, pl.ds(base, L), :],
            v_ref[0, hh, pl.ds(base, L), :],
            lambda g, hh=hh: (k_ref[0, hh, pl.ds(g, 1), :],
                              v_ref[0, hh, pl.ds(g, 1), :]),
            [g_s[hh, g] for g in range(G_PER_HEAD)],
            t0, base, S=S, T=T, L=L, D=D)


def kernel(q, k, v):
    B, H, S, D = q.shape
    P = min(PROTO_COUNT, S)
    U = max(G_PER_HEAD, min(TOP_U, S))
    idxp = np.round(np.linspace(0.0, S - 1, P)).astype(np.int32)
    Qp = _normalize_safe(q.mean(axis=0)[:, idxp, :])

    T = T_Q
    L = T + FRAG

    out = pl.pallas_call(
        functools.partial(_fused_kernel, S=S, D=D, T=T, L=L, P=P, U=U),
        grid=(H // HPB, S // T),
        in_specs=[
            pl.BlockSpec((HPB, P, D), lambda h, t: (h, 0, 0)),
            pl.BlockSpec((1, HPB, T, D), lambda h, t: (0, h, t, 0)),
            pl.BlockSpec((1, HPB, S, D), lambda h, t: (0, h, 0, 0)),
            pl.BlockSpec((1, HPB, S, D), lambda h, t: (0, h, 0, 0)),
        ],
        out_specs=pl.BlockSpec((1, HPB, T, D), lambda h, t: (0, h, t, 0)),
        out_shape=jax.ShapeDtypeStruct((B, H, S, D), jnp.float32),
        scratch_shapes=[pltpu.SMEM((HPB, GPAD), jnp.int32)],
        compiler_params=pltpu.CompilerParams(
            dimension_semantics=("parallel", "arbitrary")),
    )(Qp, q, k, v)

    return out
